# baseline (speedup 1.0000x reference)
import functools

import jax
import jax.numpy as jnp
from jax.experimental import pallas as pl
from jax.experimental.pallas import tpu as pltpu

_LANE_PAD = 128
_HEAD = 2
_OUT = 4
_INV_TEMP = 1.0 / 5.0
_TANH_C = 2.5
_CHUNK = 128
_NGATES = 4


def _ctrl_kernel(idx_ref, h_ref, c_ref, emb_ref,
                 w_ref, b_ref,
                 decw_ref, decb_ref,
                 logits_ref, hx_ref, cx_ref,
                 hbf_ref, acc_ref, *, hid, nsteps):
    n = pl.program_id(0)
    n_emb = emb_ref.shape[0]

    @pl.when(n == 0)
    def _():
        hbf_ref[...] = h_ref[...].astype(jnp.bfloat16)

    hbf = hbf_ref[...]
    embbf = emb_ref[...].astype(jnp.bfloat16)
    onehot = (idx_ref[...] == jax.lax.broadcasted_iota(
        jnp.int32, (1, n_emb), 1)).astype(jnp.bfloat16)

    gate_vals = []
    for k in range(_NGATES):
        wbf = w_ref[:, k, :].astype(jnp.bfloat16)
        eg = jnp.dot(embbf, wbf[:hid, :],
                     preferred_element_type=jnp.float32)
        eg = (eg + b_ref[k:k + 1, :]).astype(jnp.bfloat16)
        gx = jnp.dot(onehot, eg, preferred_element_type=jnp.float32)
        gh = jnp.dot(hbf, wbf[hid:, :],
                     preferred_element_type=jnp.float32)
        gate_vals.append(gx + gh)

    i_g = jax.nn.sigmoid(gate_vals[0])
    f_g = jax.nn.sigmoid(gate_vals[1])
    g_g = jnp.tanh(gate_vals[2])
    o_g = jax.nn.sigmoid(gate_vals[3])

    cx = f_g * c_ref[...] + i_g * g_g
    hx = o_g * jnp.tanh(cx)
    cx_ref[...] = cx
    hx_ref[...] = hx

    p = jnp.dot(hx.astype(jnp.bfloat16), decw_ref[...].astype(jnp.bfloat16),
                preferred_element_type=jnp.float32)

    @pl.when(n == 0)
    def _():
        acc_ref[...] = p

    @pl.when(n > 0)
    def _():
        acc_ref[...] += p

    @pl.when(n == nsteps - 1)
    def _():
        logits = acc_ref[...] + decb_ref[...]
        logits_ref[...] = (_TANH_C * jnp.tanh(logits * _INV_TEMP))[:, :_OUT]


@functools.partial(jax.jit, static_argnames=("batch", "hid"))
def _run(idx2, h0, c0, embedding, w_lstm, b_lstm, dec_w_pad, dec_b_pad,
         batch, hid):
    nsteps = hid // _CHUNK
    kernel_body = functools.partial(_ctrl_kernel, hid=hid, nsteps=nsteps)
    n_emb = embedding.shape[0]
    w4 = w_lstm.reshape(2 * hid, _NGATES, hid)
    b4 = b_lstm.reshape(_NGATES, hid)

    return pl.pallas_call(
        kernel_body,
        out_shape=(
            jax.ShapeDtypeStruct((batch, _OUT), jnp.float32),
            jax.ShapeDtypeStruct((batch, hid), jnp.float32),
            jax.ShapeDtypeStruct((batch, hid), jnp.float32),
        ),
        grid=(nsteps,),
        in_specs=[
            pl.BlockSpec((batch, 1), lambda n: (0, 0)),
            pl.BlockSpec((batch, hid), lambda n: (0, 0)),
            pl.BlockSpec((batch, _CHUNK), lambda n: (0, n)),
            pl.BlockSpec((n_emb, hid), lambda n: (0, 0)),
            pl.BlockSpec((2 * hid, _NGATES, _CHUNK),
                         lambda n: (0, 0, n)),
            pl.BlockSpec((_NGATES, _CHUNK), lambda n: (0, n)),
            pl.BlockSpec((None, _CHUNK, _LANE_PAD),
                         lambda n: (_HEAD, n, 0)),
            pl.BlockSpec((None, 1, _LANE_PAD),
                         lambda n: (_HEAD, 0, 0)),
        ],
        out_specs=(
            pl.BlockSpec((batch, _OUT), lambda n: (0, 0)),
            pl.BlockSpec((batch, _CHUNK), lambda n: (0, n)),
            pl.BlockSpec((batch, _CHUNK), lambda n: (0, n)),
        ),
        scratch_shapes=[
            pltpu.VMEM((batch, hid), jnp.bfloat16),
            pltpu.VMEM((batch, _LANE_PAD), jnp.float32),
        ],
        compiler_params=pltpu.CompilerParams(
            dimension_semantics=("arbitrary",)),
    )(idx2, h0, c0, embedding, w4, b4, dec_w_pad, dec_b_pad)


def kernel(inputs, h0, c0, embedding, w_lstm, b_lstm, dec_w_pad, dec_b_pad):
    batch = inputs.shape[0]
    hid = h0.shape[1]
    idx2 = inputs.reshape(batch, 1)
    logits, hx, cx = _run(idx2, h0, c0, embedding, w_lstm, b_lstm,
                          dec_w_pad, dec_b_pad, batch=batch, hid=hid)
    return logits, (hx, cx)


# speedup vs baseline: 1.3255x; 1.3255x over previous
import functools

import jax
import jax.numpy as jnp
from jax.experimental import pallas as pl
from jax.experimental.pallas import tpu as pltpu

_LANE_PAD = 128
_HEAD = 2
_OUT = 4
_INV_TEMP = 1.0 / 5.0
_TANH_C = 2.5
_KC = 128
_EPAD = 16


def _ctrl_kernel(idx_ref, h_ref, c_ref, emb_ref, w_ref, b_ref,
                 decw_ref, decb_ref,
                 logits_ref, hx_ref, cx_ref,
                 hbf_ref, acc_ref, eg_ref, pacc_ref,
                 *, hid, nk, nc):
    n = pl.program_id(0)
    n_emb = emb_ref.shape[0]
    nx = hid // _KC

    @pl.when(n == 0)
    def _():
        hbf_ref[...] = h_ref[...].astype(jnp.bfloat16)

    wbf = w_ref[...].astype(jnp.bfloat16)

    @pl.when(n < nx)
    def _():
        embc = emb_ref[:, pl.ds(n * _KC, _KC)].astype(jnp.bfloat16)
        embc = jnp.concatenate(
            [embc, jnp.zeros((_EPAD - n_emb, _KC), jnp.bfloat16)], axis=0)
        p = jnp.dot(embc, wbf, preferred_element_type=jnp.float32)

        @pl.when(n == 0)
        def _():
            eg_ref[...] = p

        @pl.when(n > 0)
        def _():
            eg_ref[...] += p

    @pl.when((n >= nx) & (n < nx + nk))
    def _():
        hc = hbf_ref[:, pl.ds((n - nx) * _KC, _KC)]
        p = jnp.dot(hc, wbf, preferred_element_type=jnp.float32)

        @pl.when(n == nx)
        def _():
            acc_ref[...] = p

        @pl.when(n > nx)
        def _():
            acc_ref[...] += p

    @pl.when(n >= nx + nk)
    def _():
        m = n - (nx + nk)
        onehot = (idx_ref[...] == jax.lax.broadcasted_iota(
            jnp.int32, (1, _EPAD), 1)).astype(jnp.bfloat16)

        def gate(k):
            off = k * hid + m * _CHUNK
            egb = (eg_ref[:, pl.ds(off, _CHUNK)]
                   + b_ref[:, pl.ds(off, _CHUNK)]).astype(jnp.bfloat16)
            gx = jnp.dot(onehot, egb,
                         preferred_element_type=jnp.float32)
            return gx + acc_ref[:, pl.ds(off, _CHUNK)]

        i_g = jax.nn.sigmoid(gate(0))
        f_g = jax.nn.sigmoid(gate(1))
        g_g = jnp.tanh(gate(2))
        o_g = jax.nn.sigmoid(gate(3))

        cx = f_g * c_ref[...] + i_g * g_g
        hx = o_g * jnp.tanh(cx)
        cx_ref[...] = cx
        hx_ref[...] = hx

        p = jnp.dot(hx.astype(jnp.bfloat16),
                    decw_ref[pl.ds(m * _CHUNK, _CHUNK), :].astype(jnp.bfloat16),
                    preferred_element_type=jnp.float32)

        @pl.when(m == 0)
        def _():
            pacc_ref[...] = p

        @pl.when(m > 0)
        def _():
            pacc_ref[...] += p

        @pl.when(m == nc - 1)
        def _():
            logits = pacc_ref[...] + decb_ref[...]
            logits_ref[...] = (_TANH_C * jnp.tanh(logits * _INV_TEMP))[:, :_OUT]


_CHUNK = 128


@functools.partial(jax.jit, static_argnames=("batch", "hid"))
def _run(idx2, h0, c0, embedding, w_lstm, b_lstm, dec_w_pad, dec_b_pad,
         batch, hid):
    nx = hid // _KC
    nk = hid // _KC
    nc = hid // _CHUNK
    nsteps = nx + nk + nc
    kernel_body = functools.partial(_ctrl_kernel, hid=hid, nk=nk, nc=nc)
    n_emb = embedding.shape[0]

    def cmap(n):
        m = n - (nx + nk)
        return (0, jnp.clip(m, 0, nc - 1))

    return pl.pallas_call(
        kernel_body,
        out_shape=(
            jax.ShapeDtypeStruct((batch, _OUT), jnp.float32),
            jax.ShapeDtypeStruct((batch, hid), jnp.float32),
            jax.ShapeDtypeStruct((batch, hid), jnp.float32),
        ),
        grid=(nsteps,),
        in_specs=[
            pl.BlockSpec((batch, 1), lambda n: (0, 0)),
            pl.BlockSpec((batch, hid), lambda n: (0, 0)),
            pl.BlockSpec((batch, _CHUNK), cmap),
            pl.BlockSpec((n_emb, hid), lambda n: (0, 0)),
            pl.BlockSpec((_KC, 4 * hid),
                         lambda n: (jnp.clip(n, 0, 2 * hid // _KC - 1), 0)),
            pl.BlockSpec((1, 4 * hid), lambda n: (0, 0)),
            pl.BlockSpec((None, hid, _LANE_PAD),
                         lambda n: (_HEAD, 0, 0)),
            pl.BlockSpec((None, 1, _LANE_PAD),
                         lambda n: (_HEAD, 0, 0)),
        ],
        out_specs=(
            pl.BlockSpec((batch, _OUT), lambda n: (0, 0)),
            pl.BlockSpec((batch, _CHUNK), cmap),
            pl.BlockSpec((batch, _CHUNK), cmap),
        ),
        scratch_shapes=[
            pltpu.VMEM((batch, hid), jnp.bfloat16),
            pltpu.VMEM((batch, 4 * hid), jnp.float32),
            pltpu.VMEM((_EPAD, 4 * hid), jnp.float32),
            pltpu.VMEM((batch, _LANE_PAD), jnp.float32),
        ],
        compiler_params=pltpu.CompilerParams(
            dimension_semantics=("arbitrary",)),
    )(idx2, h0, c0, embedding, w_lstm, b_lstm, dec_w_pad, dec_b_pad)


def kernel(inputs, h0, c0, embedding, w_lstm, b_lstm, dec_w_pad, dec_b_pad):
    batch = inputs.shape[0]
    hid = h0.shape[1]
    idx2 = inputs.reshape(batch, 1)
    logits, hx, cx = _run(idx2, h0, c0, embedding, w_lstm, b_lstm,
                          dec_w_pad, dec_b_pad, batch=batch, hid=hid)
    return logits, (hx, cx)


# speedup vs baseline: 1.8698x; 1.4107x over previous
import functools

import jax
import jax.numpy as jnp
from jax.experimental import pallas as pl
from jax.experimental.pallas import tpu as pltpu

_LANE_PAD = 128
_HEAD = 2
_OUT = 4
_INV_TEMP = 1.0 / 5.0
_TANH_C = 2.5
_KC = 128
_NB = 4
_EPAD = 16


def _ctrl_kernel(idx_ref, h_ref, c_ref, emb_ref, w_ref, b_ref,
                 decw_ref, decb_ref,
                 logits_ref, hx_ref, cx_ref,
                 wsc_ref, *, hid, nk):
    n = pl.program_id(0)
    n_emb = emb_ref.shape[0]

    wsc_ref[pl.ds(jnp.clip(n, 0, nk - 1) * _KC, _KC), :] = (
        w_ref[...].astype(jnp.bfloat16))

    @pl.when(n >= nk)
    def _():
        hbf = h_ref[...].astype(jnp.bfloat16)

        embp = jnp.concatenate(
            [emb_ref[...].astype(jnp.bfloat16),
             jnp.zeros((_EPAD - n_emb, hid), jnp.bfloat16)], axis=0)
        eg = jnp.dot(embp, wsc_ref[:hid, :],
                     preferred_element_type=jnp.float32)
        eg = (eg + b_ref[...]).astype(jnp.bfloat16)
        onehot = (idx_ref[...] == jax.lax.broadcasted_iota(
            jnp.int32, (1, _EPAD), 1)).astype(jnp.bfloat16)
        gx = jnp.dot(onehot, eg, preferred_element_type=jnp.float32)

        gh = jnp.dot(hbf, wsc_ref[hid:, :],
                     preferred_element_type=jnp.float32)
        gates = gx + gh

        i_g = jax.nn.sigmoid(gates[:, 0 * hid:1 * hid])
        f_g = jax.nn.sigmoid(gates[:, 1 * hid:2 * hid])
        g_g = jnp.tanh(gates[:, 2 * hid:3 * hid])
        o_g = jax.nn.sigmoid(gates[:, 3 * hid:4 * hid])

        cx = f_g * c_ref[...] + i_g * g_g
        hx = o_g * jnp.tanh(cx)
        cx_ref[...] = cx
        hx_ref[...] = hx

        logits = (jnp.dot(hx.astype(jnp.bfloat16),
                          decw_ref[...].astype(jnp.bfloat16),
                          preferred_element_type=jnp.float32)
                  + decb_ref[...])
        logits_ref[...] = (_TANH_C * jnp.tanh(logits * _INV_TEMP))[:, :_OUT]


@functools.partial(jax.jit, static_argnames=("batch", "hid"))
def _run(idx2, h0, c0, embedding, w_lstm, b_lstm, dec_w_pad, dec_b_pad,
         batch, hid):
    nk = 2 * hid // _KC
    bt = batch // _NB
    nsteps = nk + _NB
    kernel_body = functools.partial(_ctrl_kernel, hid=hid, nk=nk)
    n_emb = embedding.shape[0]

    def bmap(n):
        return (jnp.clip(n - nk, 0, _NB - 1), 0)

    return pl.pallas_call(
        kernel_body,
        out_shape=(
            jax.ShapeDtypeStruct((batch, _OUT), jnp.float32),
            jax.ShapeDtypeStruct((batch, hid), jnp.float32),
            jax.ShapeDtypeStruct((batch, hid), jnp.float32),
        ),
        grid=(nsteps,),
        in_specs=[
            pl.BlockSpec((bt, 1), bmap),
            pl.BlockSpec((bt, hid), bmap),
            pl.BlockSpec((bt, hid), bmap),
            pl.BlockSpec((n_emb, hid), lambda n: (0, 0)),
            pl.BlockSpec((_KC, 4 * hid),
                         lambda n: (jnp.clip(n, 0, 2 * hid // _KC - 1), 0)),
            pl.BlockSpec((1, 4 * hid), lambda n: (0, 0)),
            pl.BlockSpec((None, hid, _LANE_PAD),
                         lambda n: (_HEAD, 0, 0)),
            pl.BlockSpec((None, 1, _LANE_PAD),
                         lambda n: (_HEAD, 0, 0)),
        ],
        out_specs=(
            pl.BlockSpec((bt, _OUT), bmap),
            pl.BlockSpec((bt, hid), bmap),
            pl.BlockSpec((bt, hid), bmap),
        ),
        scratch_shapes=[
            pltpu.VMEM((2 * hid, 4 * hid), jnp.bfloat16),
        ],
        compiler_params=pltpu.CompilerParams(
            dimension_semantics=("arbitrary",)),
    )(idx2, h0, c0, embedding, w_lstm, b_lstm, dec_w_pad, dec_b_pad)


def kernel(inputs, h0, c0, embedding, w_lstm, b_lstm, dec_w_pad, dec_b_pad):
    batch = inputs.shape[0]
    hid = h0.shape[1]
    idx2 = inputs.reshape(batch, 1)
    logits, hx, cx = _run(idx2, h0, c0, embedding, w_lstm, b_lstm,
                          dec_w_pad, dec_b_pad, batch=batch, hid=hid)
    return logits, (hx, cx)


# speedup vs baseline: 2.0988x; 1.1225x over previous
import functools

import jax
import jax.numpy as jnp
from jax.experimental import pallas as pl
from jax.experimental.pallas import tpu as pltpu

_LANE_PAD = 128
_HEAD = 2
_OUT = 4
_INV_TEMP = 1.0 / 5.0
_TANH_C = 2.5
_NB = 2
_EPAD = 16


def _sigmoid(x):
    return 0.5 * jnp.tanh(0.5 * x) + 0.5


def _ctrl_kernel(idx_ref, h_ref, c_ref, emb_ref, w_ref, b_ref,
                 decw_ref, decb_ref,
                 logits_ref, hx_ref, cx_ref,
                 wsc_ref, egb_ref, dsc_ref, *, hid):
    j = pl.program_id(1)
    n_emb = emb_ref.shape[0]

    @pl.when(j == 0)
    def _():
        wsc_ref[...] = w_ref[...].astype(jnp.bfloat16)
        dsc_ref[...] = decw_ref[...].astype(jnp.bfloat16)
        embp = jnp.concatenate(
            [emb_ref[...].astype(jnp.bfloat16),
             jnp.zeros((_EPAD - n_emb, hid), jnp.bfloat16)], axis=0)
        eg = jnp.dot(embp, wsc_ref[:hid, :],
                     preferred_element_type=jnp.float32)
        egb_ref[...] = (eg + b_ref[...]).astype(jnp.bfloat16)

    onehot = (idx_ref[...] == jax.lax.broadcasted_iota(
        jnp.int32, (1, _EPAD), 1)).astype(jnp.bfloat16)
    gx = jnp.dot(onehot, egb_ref[...], preferred_element_type=jnp.float32)
    gh = jnp.dot(h_ref[...].astype(jnp.bfloat16), wsc_ref[hid:, :],
                 preferred_element_type=jnp.float32)
    gates = gx + gh

    i_g = _sigmoid(gates[:, 0 * hid:1 * hid])
    f_g = _sigmoid(gates[:, 1 * hid:2 * hid])
    g_g = jnp.tanh(gates[:, 2 * hid:3 * hid])
    o_g = _sigmoid(gates[:, 3 * hid:4 * hid])

    cx = f_g * c_ref[...] + i_g * g_g
    hx = o_g * jnp.tanh(cx)
    cx_ref[...] = cx
    hx_ref[...] = hx

    logits = (jnp.dot(hx.astype(jnp.bfloat16), dsc_ref[...],
                      preferred_element_type=jnp.float32)
              + decb_ref[...])
    logits_ref[...] = (_TANH_C * jnp.tanh(logits * _INV_TEMP))[:, :_OUT]


@functools.partial(jax.jit, static_argnames=("batch", "hid"))
def _run(idx2, h0, c0, embedding, w_lstm, b_lstm, dec_w_pad, dec_b_pad,
         batch, hid):
    bt = batch // (2 * _NB)
    kernel_body = functools.partial(_ctrl_kernel, hid=hid)
    n_emb = embedding.shape[0]

    def bmap(i, j):
        return (i * _NB + j, 0)

    return pl.pallas_call(
        kernel_body,
        out_shape=(
            jax.ShapeDtypeStruct((batch, _OUT), jnp.float32),
            jax.ShapeDtypeStruct((batch, hid), jnp.float32),
            jax.ShapeDtypeStruct((batch, hid), jnp.float32),
        ),
        grid=(2, _NB),
        in_specs=[
            pl.BlockSpec((bt, 1), bmap),
            pl.BlockSpec((bt, hid), bmap),
            pl.BlockSpec((bt, hid), bmap),
            pl.BlockSpec((n_emb, hid), lambda i, j: (0, 0)),
            pl.BlockSpec((2 * hid, 4 * hid), lambda i, j: (0, 0)),
            pl.BlockSpec((1, 4 * hid), lambda i, j: (0, 0)),
            pl.BlockSpec((None, hid, _LANE_PAD),
                         lambda i, j: (_HEAD, 0, 0)),
            pl.BlockSpec((None, 1, _LANE_PAD),
                         lambda i, j: (_HEAD, 0, 0)),
        ],
        out_specs=(
            pl.BlockSpec((bt, _OUT), bmap),
            pl.BlockSpec((bt, hid), bmap),
            pl.BlockSpec((bt, hid), bmap),
        ),
        scratch_shapes=[
            pltpu.VMEM((2 * hid, 4 * hid), jnp.bfloat16),
            pltpu.VMEM((_EPAD, 4 * hid), jnp.bfloat16),
            pltpu.VMEM((hid, _LANE_PAD), jnp.bfloat16),
        ],
        compiler_params=pltpu.CompilerParams(
            dimension_semantics=("parallel", "arbitrary")),
    )(idx2, h0, c0, embedding, w_lstm, b_lstm, dec_w_pad, dec_b_pad)


def kernel(inputs, h0, c0, embedding, w_lstm, b_lstm, dec_w_pad, dec_b_pad):
    batch = inputs.shape[0]
    hid = h0.shape[1]
    idx2 = inputs.reshape(batch, 1)
    logits, hx, cx = _run(idx2, h0, c0, embedding, w_lstm, b_lstm,
                          dec_w_pad, dec_b_pad, batch=batch, hid=hid)
    return logits, (hx, cx)


# speedup vs baseline: 2.1139x; 1.0072x over previous
import functools

import jax
import jax.numpy as jnp
from jax.experimental import pallas as pl
from jax.experimental.pallas import tpu as pltpu

_LANE_PAD = 128
_HEAD = 2
_OUT = 4
_INV_TEMP = 1.0 / 5.0
_TANH_C = 2.5
_BT = 384


def _ctrl_kernel(idx_ref, h_ref, c_ref, emb_ref, w_ref, b_ref,
                 decw_ref, decb_ref, logits_ref, hx_ref, cx_ref, *, hid):
    n_emb = emb_ref.shape[0]
    wx = w_ref[:hid, :].astype(jnp.bfloat16)
    wh = w_ref[hid:, :].astype(jnp.bfloat16)

    eg = jnp.dot(emb_ref[...].astype(jnp.bfloat16), wx,
                 preferred_element_type=jnp.float32)
    eg = (eg + b_ref[...]).astype(jnp.bfloat16)
    onehot = (idx_ref[...] == jax.lax.broadcasted_iota(
        jnp.int32, (1, n_emb), 1)).astype(jnp.bfloat16)
    gx = jnp.dot(onehot, eg, preferred_element_type=jnp.float32)

    gh = jnp.dot(h_ref[...].astype(jnp.bfloat16), wh,
                 preferred_element_type=jnp.float32)
    gates = gx + gh

    i_g = jax.nn.sigmoid(gates[:, 0 * hid:1 * hid])
    f_g = jax.nn.sigmoid(gates[:, 1 * hid:2 * hid])
    g_g = jnp.tanh(gates[:, 2 * hid:3 * hid])
    o_g = jax.nn.sigmoid(gates[:, 3 * hid:4 * hid])

    cx = f_g * c_ref[...] + i_g * g_g
    hx = o_g * jnp.tanh(cx)

    logits = (jnp.dot(hx.astype(jnp.bfloat16),
                      decw_ref[...].astype(jnp.bfloat16),
                      preferred_element_type=jnp.float32)
              + decb_ref[...])
    logits = _TANH_C * jnp.tanh(logits * _INV_TEMP)
    logits_ref[...] = logits[:, :_OUT]
    hx_ref[...] = hx
    cx_ref[...] = cx


@functools.partial(jax.jit, static_argnames=("batch", "hid", "bt"))
def _run(idx2, h0, c0, embedding, w_lstm, b_lstm, dec_w_pad, dec_b_pad,
         batch, hid, bt):
    kernel_body = functools.partial(_ctrl_kernel, hid=hid)
    n_emb = embedding.shape[0]
    return pl.pallas_call(
        kernel_body,
        out_shape=(
            jax.ShapeDtypeStruct((batch, _OUT), jnp.float32),
            jax.ShapeDtypeStruct((batch, hid), jnp.float32),
            jax.ShapeDtypeStruct((batch, hid), jnp.float32),
        ),
        grid=(batch // bt,),
        in_specs=[
            pl.BlockSpec((bt, 1), lambda i: (i, 0)),
            pl.BlockSpec((bt, hid), lambda i: (i, 0)),
            pl.BlockSpec((bt, hid), lambda i: (i, 0)),
            pl.BlockSpec((n_emb, hid), lambda i: (0, 0)),
            pl.BlockSpec((2 * hid, 4 * hid), lambda i: (0, 0)),
            pl.BlockSpec((1, 4 * hid), lambda i: (0, 0)),
            pl.BlockSpec((None, hid, _LANE_PAD),
                         lambda i: (_HEAD, 0, 0)),
            pl.BlockSpec((None, 1, _LANE_PAD),
                         lambda i: (_HEAD, 0, 0)),
        ],
        out_specs=(
            pl.BlockSpec((bt, _OUT), lambda i: (i, 0)),
            pl.BlockSpec((bt, hid), lambda i: (i, 0)),
            pl.BlockSpec((bt, hid), lambda i: (i, 0)),
        ),
        compiler_params=pltpu.CompilerParams(
            dimension_semantics=("parallel",)),
    )(idx2, h0, c0, embedding, w_lstm, b_lstm, dec_w_pad, dec_b_pad)


def kernel(inputs, h0, c0, embedding, w_lstm, b_lstm, dec_w_pad, dec_b_pad):
    batch = inputs.shape[0]
    hid = h0.shape[1]

    bt = _BT
    while batch % bt:
        bt //= 2

    idx2 = inputs.reshape(batch, 1)
    logits, hx, cx = _run(idx2, h0, c0, embedding, w_lstm, b_lstm,
                          dec_w_pad, dec_b_pad,
                          batch=batch, hid=hid, bt=bt)
    return logits, (hx, cx)
